# baseline (speedup 1.0000x reference)
import functools

import jax
import jax.numpy as jnp
from jax import lax
from jax.experimental import pallas as pl
from jax.experimental.pallas import tpu as pltpu
from jax.experimental.pallas import tpu_sc as plsc

NC = 2
NS = 16
L = 16
NW = NC * NS
BATCH = 16384
BPW = BATCH // NW
D = 32
CHUNK = 128
NCHUNK = BPW // CHUNK


def _body(x_hbm, w_hbm, h_hbm, out_hbm,
          xv, uidx, vidx, urows, vrows, res, sem_u, sem_v):
    cid = lax.axis_index("c")
    sid = lax.axis_index("s")
    wid = sid * NC + cid
    base = wid * BPW

    pltpu.sync_copy(x_hbm.at[pl.ds(base * 2, BPW * 2)], xv)

    lanes = lax.iota(jnp.int32, L)

    def deint(j, _):
        pos = jnp.full((L,), 2 * j * L, jnp.int32) + 2 * lanes
        u = plsc.load_gather(xv, [pos])
        v = plsc.load_gather(xv, [pos + 1])
        k = j // (CHUNK // L)
        o = (j % (CHUNK // L)) * L
        uidx[k, pl.ds(o, L)] = u
        vidx[k, pl.ds(o, L)] = v
        return 0

    lax.fori_loop(0, BPW // L, deint, 0, unroll=2)

    copies = []
    for k in range(NCHUNK):
        copies.append(pltpu.async_copy(
            w_hbm.at[uidx.at[k]], urows.at[pl.ds(k * CHUNK, CHUNK)], sem_u))
        copies.append(pltpu.async_copy(
            h_hbm.at[vidx.at[k]], vrows.at[pl.ds(k * CHUNK, CHUNK)], sem_v))
    for cp in copies:
        cp.wait()

    def group(g, _):
        row = jnp.full((L,), g * L, jnp.int32) + lanes

        def col(c, acc):
            cv = (jnp.full((L,), c, jnp.int32) + lanes) & (D - 1)
            u = plsc.load_gather(urows, [row, cv])
            v = plsc.load_gather(vrows, [row, cv])
            return acc + u * v

        acc = lax.fori_loop(0, D, col, jnp.zeros((L,), jnp.float32),
                            unroll=4)
        res[pl.ds(g * L, L)] = 1.0 / (1.0 + jnp.exp(-acc))
        return 0

    lax.fori_loop(0, BPW // L, group, 0)

    pltpu.sync_copy(res, out_hbm.at[pl.ds(base, BPW)])


@jax.jit
def kernel(x, W, H):
    mesh = plsc.VectorSubcoreMesh(
        core_axis_name="c", subcore_axis_name="s",
        num_cores=NC, num_subcores=NS)
    run = pl.kernel(
        _body,
        out_type=jax.ShapeDtypeStruct((BATCH,), jnp.float32),
        mesh=mesh,
        compiler_params=pltpu.CompilerParams(
            needs_layout_passes=False, use_tc_tiling_on_sc=False),
        scratch_types=[
            pltpu.VMEM((BPW * 2,), jnp.int32),
            pltpu.VMEM((NCHUNK, CHUNK), jnp.int32),
            pltpu.VMEM((NCHUNK, CHUNK), jnp.int32),
            pltpu.VMEM((BPW, D), jnp.float32),
            pltpu.VMEM((BPW, D), jnp.float32),
            pltpu.VMEM((BPW,), jnp.float32),
            pltpu.SemaphoreType.DMA,
            pltpu.SemaphoreType.DMA,
        ],
    )
    return run(x.reshape(-1), W, H)
